# Initial kernel scaffold; baseline (speedup 1.0000x reference)
import jax
import jax.numpy as jnp
from jax.experimental import pallas as pl


def kernel(gene_ids, table):
    raise NotImplementedError("write your pallas kernel here")


# speedup vs baseline: 3.5349x; 3.5349x over previous
import functools

import jax
import jax.numpy as jnp
from jax import lax
from jax.experimental import pallas as pl
from jax.experimental.pallas import tpu as pltpu
from jax.experimental.pallas import tpu_sc as plsc

EMBED_DIM = 64
CHUNK = 128


@functools.lru_cache(maxsize=None)
def _make_gather(n_idx: int, vocab: int, d: int):
    info = plsc.get_sparse_core_info()
    nc, ns = info.num_cores, info.num_subcores
    nw = nc * ns
    assert n_idx % (nw * CHUNK) == 0
    steps = n_idx // (nw * CHUNK)

    @functools.partial(
        pl.kernel,
        mesh=plsc.VectorSubcoreMesh(core_axis_name="c", subcore_axis_name="s"),
        out_type=jax.ShapeDtypeStruct((n_idx, d), jnp.float32),
        scratch_types=[
            pltpu.VMEM((steps, CHUNK), jnp.int32),
            pltpu.VMEM((CHUNK, d), jnp.float32),
            pltpu.SemaphoreType.DMA,
        ],
        compiler_params=pltpu.CompilerParams(use_tc_tiling_on_sc=False),
    )
    def gather_kernel(idx_hbm, table_hbm, out_hbm, idx_v, rows_v, sem):
        wid = lax.axis_index("s") * nc + lax.axis_index("c")
        base = wid * (steps * CHUNK)
        pltpu.sync_copy(idx_hbm.at[wid], idx_v)

        def step(j, carry):
            pltpu.async_copy(table_hbm.at[idx_v.at[j]], rows_v, sem).wait()
            pltpu.sync_copy(rows_v, out_hbm.at[pl.ds(base + j * CHUNK, CHUNK)])
            return carry

        lax.fori_loop(0, steps, step, 0)

    return gather_kernel


def kernel(gene_ids, table):
    b, s = gene_ids.shape
    vocab, d = table.shape
    info = plsc.get_sparse_core_info()
    nw = info.num_cores * info.num_subcores
    n_idx = b * s
    idx = gene_ids.reshape(nw, n_idx // (nw * CHUNK), CHUNK).astype(jnp.int32)
    out = _make_gather(n_idx, vocab, d)(idx, table)
    return gene_ids, out.reshape(b, s, d)


# speedup vs baseline: 4.2344x; 1.1979x over previous
import functools

import jax
import jax.numpy as jnp
from jax import lax
from jax.experimental import pallas as pl
from jax.experimental.pallas import tpu as pltpu
from jax.experimental.pallas import tpu_sc as plsc

EMBED_DIM = 64
CHUNK = 128
K = 5


@functools.lru_cache(maxsize=None)
def _make_gather(n_idx: int, vocab: int, d: int):
    info = plsc.get_sparse_core_info()
    nc, ns = info.num_cores, info.num_subcores
    nw = nc * ns
    assert n_idx % (nw * CHUNK * K) == 0
    steps = n_idx // (nw * CHUNK)
    groups = steps // K
    grows = K * CHUNK

    @functools.partial(
        pl.kernel,
        mesh=plsc.VectorSubcoreMesh(core_axis_name="c", subcore_axis_name="s"),
        out_type=jax.ShapeDtypeStruct((n_idx, d), jnp.float32),
        scratch_types=[
            pltpu.VMEM((steps, CHUNK), jnp.int32),
            pltpu.VMEM((2, grows, d), jnp.float32),
            pltpu.SemaphoreType.DMA,
            pltpu.SemaphoreType.DMA,
        ],
        compiler_params=pltpu.CompilerParams(use_tc_tiling_on_sc=False),
    )
    def gather_kernel(idx_hbm, table_hbm, out_hbm, idx_v, rows_v, gsem, osem):
        wid = lax.axis_index("s") * nc + lax.axis_index("c")
        base = wid * (steps * CHUNK)
        pltpu.sync_copy(idx_hbm.at[wid], idx_v)

        def fire(g, p):
            return [
                pltpu.async_copy(
                    table_hbm.at[idx_v.at[g * K + j]],
                    rows_v.at[p].at[pl.ds(j * CHUNK, CHUNK)],
                    gsem,
                )
                for j in range(K)
            ]

        def start_out(g, p):
            return pltpu.async_copy(
                rows_v.at[p], out_hbm.at[pl.ds(base + g * grows, grows)], osem
            )

        for d_ in fire(0, 0):
            d_.wait()

        def body(i, carry):
            p = i % 2
            od = start_out(i, p)
            gds = fire(i + 1, 1 - p)
            for d_ in gds:
                d_.wait()
            od.wait()
            return carry

        lax.fori_loop(0, groups - 1, body, 0)
        start_out(groups - 1, (groups - 1) % 2).wait()

    return gather_kernel


def kernel(gene_ids, table):
    b, s = gene_ids.shape
    vocab, d = table.shape
    info = plsc.get_sparse_core_info()
    nw = info.num_cores * info.num_subcores
    n_idx = b * s
    idx = gene_ids.reshape(nw, n_idx // (nw * CHUNK), CHUNK).astype(jnp.int32)
    out = _make_gather(n_idx, vocab, d)(idx, table)
    return gene_ids, out.reshape(b, s, d)
